# baseline (speedup 1.0000x reference)
import functools

import jax
import jax.numpy as jnp
from jax import lax
from jax.experimental import pallas as pl
from jax.experimental.pallas import tpu as pltpu
from jax.experimental.pallas import tpu_sc as plsc

N_NODES = 40962
N_OUT = 10242
K = 7
N_ROWS = 1024
NUM_WORKERS = 32
ROWS_PER_W = N_ROWS // NUM_WORKERS
ROW_TILES = (((N_NODES + 127) // 128 + 7) // 8) * 8
ROW_PAD = ROW_TILES * 128
OUT_TILES = (((N_OUT + 127) // 128 + 7) // 8) * 8
OUT_STRIDE = OUT_TILES * 128
VAL_TILES = (N_OUT + 127) // 128
GROUPS = (VAL_TILES * 128) // 16
IDX_TILES = ((K * VAL_TILES + 7) // 8) * 8


@functools.partial(
    pl.kernel,
    mesh=plsc.VectorSubcoreMesh(core_axis_name="c", subcore_axis_name="s"),
    compiler_params=pltpu.CompilerParams(needs_layout_passes=False),
    out_type=jax.ShapeDtypeStruct((N_ROWS * OUT_TILES, 128), jnp.float32),
    scratch_types=[
        pltpu.VMEM((IDX_TILES, 128), jnp.int32),
        pltpu.VMEM((ROW_TILES, 128), jnp.float32),
        pltpu.VMEM((OUT_TILES, 128), jnp.float32),
    ],
)
def _pool(x_hbm, idx_hbm, out_hbm, idx_v, row_v, out_v):
    wid = lax.axis_index("s") * 2 + lax.axis_index("c")
    pltpu.sync_copy(idx_hbm, idx_v)
    scale = jnp.float32(1.0 / K)

    def row_step(r, carry):
        row = wid * ROWS_PER_W + r
        pltpu.sync_copy(x_hbm.at[pl.ds(row * ROW_TILES, ROW_TILES)], row_v)

        def grp(g, c2):
            q = g // 8
            m = (g % 8) * 16
            vals = []
            for k in range(K):
                ivec = idx_v[k * VAL_TILES + q, pl.ds(m, 16)]
                vals.append(
                    plsc.load_gather(row_v, [ivec >> 7, ivec & 127])
                )
            s01 = vals[0] + vals[1]
            s23 = vals[2] + vals[3]
            s45 = vals[4] + vals[5]
            acc = (s01 + s23) + (s45 + vals[6])
            out_v[q, pl.ds(m, 16)] = acc * scale
            return c2

        lax.fori_loop(0, GROUPS, grp, 0)
        pltpu.sync_copy(out_v, out_hbm.at[pl.ds(row * OUT_TILES, OUT_TILES)])
        return carry

    lax.fori_loop(0, ROWS_PER_W, row_step, 0)


def kernel(x, neigh_orders):
    B, D, N = x.shape
    idx = neigh_orders[: N_OUT * K].astype(jnp.int32).reshape(N_OUT, K).T
    idx = jnp.pad(idx, ((0, 0), (0, VAL_TILES * 128 - N_OUT))).reshape(-1)
    idx = jnp.pad(idx, (0, IDX_TILES * 128 - idx.shape[0])).reshape(IDX_TILES, 128)
    xp = jnp.pad(x.reshape(B * D, N), ((0, 0), (0, ROW_PAD - N)))
    xp = xp.reshape(N_ROWS * ROW_TILES, 128)
    out = _pool(xp, idx)
    out = out.reshape(N_ROWS, OUT_STRIDE)[:, :N_OUT]
    return out.reshape(B, D, N_OUT)


# speedup vs baseline: 1.2694x; 1.2694x over previous
import functools

import jax
import jax.numpy as jnp
from jax import lax
from jax.experimental import pallas as pl
from jax.experimental.pallas import tpu as pltpu
from jax.experimental.pallas import tpu_sc as plsc

N_NODES = 40962
N_OUT = 10242
K = 7
N_ROWS = 1024
NUM_WORKERS = 32
ROWS_PER_W = N_ROWS // NUM_WORKERS
ROW_TILES = (((N_NODES + 127) // 128 + 7) // 8) * 8
ROW_PAD = ROW_TILES * 128
OUT_TILES = (((N_OUT + 127) // 128 + 7) // 8) * 8
OUT_STRIDE = OUT_TILES * 128
VAL_TILES = (N_OUT + 127) // 128
PKQ = 4 * K * 16
PKQ_PAD = 512
PK_ROWS = ((VAL_TILES * 4 + 7) // 8) * 8


@functools.partial(
    pl.kernel,
    mesh=plsc.VectorSubcoreMesh(core_axis_name="c", subcore_axis_name="s"),
    compiler_params=pltpu.CompilerParams(needs_layout_passes=False),
    out_type=jax.ShapeDtypeStruct((N_ROWS * OUT_TILES, 128), jnp.float32),
    scratch_types=[
        pltpu.VMEM((PK_ROWS, 128), jnp.float32),
        pltpu.VMEM((ROW_TILES, 128), jnp.float32),
        pltpu.VMEM((OUT_TILES, 128), jnp.float32),
    ],
)
def _pool(x_hbm, idx_hbm, out_hbm, idx_v, row_v, out_v):
    wid = lax.axis_index("s") * 2 + lax.axis_index("c")
    pltpu.sync_copy(idx_hbm, idx_v)
    scale = jnp.float32(1.0 / K)
    m16 = jnp.uint32(0xFFFF)
    m7 = jnp.uint32(127)

    def row_step(r, carry):
        row = wid * ROWS_PER_W + r
        pltpu.sync_copy(x_hbm.at[pl.ds(row * ROW_TILES, ROW_TILES)], row_v)

        def tile_q(q, c2):
            base = q * 4
            acc = [None] * 8
            for k in range(K):
                for h in range(4):
                    off = 64 * k + 16 * h
                    vecf = idx_v[base + off // 128, pl.ds(off % 128, 16)]
                    w = plsc.bitcast(vecf, jnp.uint32)
                    a = w & m16
                    b = w >> 16
                    va = plsc.load_gather(
                        row_v,
                        [plsc.bitcast(a >> 7, jnp.int32),
                         plsc.bitcast(a & m7, jnp.int32)],
                    )
                    vb = plsc.load_gather(
                        row_v,
                        [plsc.bitcast(b >> 7, jnp.int32),
                         plsc.bitcast(b & m7, jnp.int32)],
                    )
                    ia, ib = 2 * h, 2 * h + 1
                    if k == 0:
                        acc[ia], acc[ib] = va, vb
                    else:
                        acc[ia] = acc[ia] + va
                        acc[ib] = acc[ib] + vb
            for h in range(4):
                out_v[q, pl.ds(32 * h, 16)] = acc[2 * h] * scale
                out_v[q, pl.ds(32 * h + 16, 16)] = acc[2 * h + 1] * scale
            return c2

        lax.fori_loop(0, VAL_TILES, tile_q, 0)
        pltpu.sync_copy(out_v, out_hbm.at[pl.ds(row * OUT_TILES, OUT_TILES)])
        return carry

    lax.fori_loop(0, ROWS_PER_W, row_step, 0)


def _pack_indices(neigh_orders):
    idx = neigh_orders[: N_OUT * K].astype(jnp.int32).reshape(N_OUT, K).T
    idx = jnp.pad(idx, ((0, 0), (0, VAL_TILES * 128 - N_OUT)))
    a = idx.reshape(K, VAL_TILES, 4, 2, 16)
    packed = a[:, :, :, 0, :] | (a[:, :, :, 1, :] << 16)
    packed = packed.transpose(1, 0, 2, 3).reshape(VAL_TILES, PKQ)
    packed = jnp.pad(packed, ((0, 0), (0, PKQ_PAD - PKQ))).reshape(-1)
    packed = jnp.pad(packed, (0, PK_ROWS * 128 - packed.shape[0]))
    return lax.bitcast_convert_type(packed.reshape(PK_ROWS, 128), jnp.float32)


def kernel(x, neigh_orders):
    B, D, N = x.shape
    idx = _pack_indices(neigh_orders)
    xp = jnp.pad(x.reshape(B * D, N), ((0, 0), (0, ROW_PAD - N)))
    xp = xp.reshape(N_ROWS * ROW_TILES, 128)
    out = _pool(xp, idx)
    out = out.reshape(N_ROWS, OUT_STRIDE)[:, :N_OUT]
    return out.reshape(B, D, N_OUT)
